# baseline (speedup 1.0000x reference)
import jax
import jax.numpy as jnp
from jax import lax
from jax.experimental import pallas as pl
from jax.experimental.pallas import tpu as pltpu
from jax.experimental.pallas import tpu_sc as plsc

_B, _C, _N, _E = 16, 7, 4096, 32004
_ND, _ED, _H = 7, 5, 64
_BP = _B // 2
_H2 = 2 * _H
_EPAD = 32768
_EBLK = 1024
_EB = _EPAD // _EBLK
_NBLK = 1024
_NB = _N // _NBLK

_EHW = _EPAD // 4
_GK = 4
_GG = _EHW // (128 * _GK)
_EPW = _EPAD // 16
_SK = 4
_SG = _EPW // (128 * _SK)
_NPT = _N // 16


def _gelu(x):
    return 0.5 * x * (1.0 + jax.lax.erf(x * 0.7071067811865476))


def _ln(z, g, b):
    mu = jnp.mean(z, axis=-1, keepdims=True)
    var = jnp.mean((z - mu) ** 2, axis=-1, keepdims=True)
    return (z - mu) / jnp.sqrt(var + 1e-5) * g + b


def _ln2(z, g, b):
    return jnp.concatenate(
        [_ln(z[:, :_H], g[:, :_H], b[:, :_H]),
         _ln(z[:, _H:], g[:, _H:], b[:, _H:])], axis=-1)


def _stage1(x_ref, wn_ref, bn_ref, g1_ref, b1_ref, h_ref):
    z = jnp.dot(x_ref[0], wn_ref[...], preferred_element_type=jnp.float32)
    z = z + bn_ref[...]
    h_ref[...] = _gelu(_ln2(z, g1_ref[...], b1_ref[...]))


def _sc_gather(h0_hbm, src_hbm, out_hbm, idx_v, rows_v, gsem, osem):
    c = lax.axis_index("c")
    s = lax.axis_index("s")
    w = s * 2 + c
    p = w // 4
    quarter = w % 4
    base_e = quarter * _EHW
    pltpu.sync_copy(src_hbm.at[pl.ds(base_e, _EHW)], idx_v)
    pn = p * _N

    def _addbase(i, _):
        idx_v[pl.ds(i * 16, 16)] = idx_v[pl.ds(i * 16, 16)] + pn
        return 0

    lax.fori_loop(0, _EHW // 16, _addbase, 0)

    def _group(g, _):
        hs = []
        for k in range(_GK):
            j = g * _GK + k
            hs.append(pltpu.async_copy(
                h0_hbm.at[idx_v.at[pl.ds(j * 128, 128)]], rows_v.at[k], gsem))
        for k in range(_GK):
            hs[k].wait()
        os = []
        for k in range(_GK):
            j = g * _GK + k
            os.append(pltpu.async_copy(
                rows_v.at[k],
                out_hbm.at[p, pl.ds(base_e + j * 128, 128), :], osem))
        for k in range(_GK):
            os[k].wait()
        return 0

    lax.fori_loop(0, _GG, _group, 0)


def _stage2(ea_ref, hs_ref, pe_ref, ge_ref,
            we_ref, be_ref, g2_ref, b2_ref, wmt_ref, wmb_ref, bm_ref,
            m_ref, nume_ref, dene_ref, invp, an0, ad0, an1, ad1):
    j = pl.program_id(0)
    b = pl.program_id(1)
    z = jnp.dot(ea_ref[0], we_ref[...], preferred_element_type=jnp.float32)
    z = z + be_ref[...]
    e = _gelu(_ln2(z, g2_ref[...], b2_ref[...]))
    mpre = (jnp.dot(hs_ref[0], wmt_ref[...], preferred_element_type=jnp.float32)
            + jnp.dot(e, wmb_ref[...], preferred_element_type=jnp.float32)
            + bm_ref[...])
    m = _gelu(mpre)
    validc = jax.lax.broadcasted_iota(jnp.int32, (_EBLK, 1), 0) < (_E - j * _EBLK)
    m_ref[0] = jnp.where(validc, m, 0.0)

    pe = pe_ref[...]

    @pl.when(b == 0)
    def _():
        sp = jnp.sum(pe * pe, axis=-1)
        invp[...] = 1.0 / jnp.maximum(jnp.sqrt(sp), 1e-6)

    ges = jax.nn.sigmoid(ge_ref[...])
    validr = jax.lax.broadcasted_iota(
        jnp.int32, (1, _EBLK), 1) < (_E - j * _EBLK)

    e0 = e[:, :_H]
    e1 = e[:, _H:]
    d0 = jnp.sum(pe * e0[None], axis=-1)
    d1 = jnp.sum(pe * e1[None], axis=-1)
    inve0 = 1.0 / jnp.maximum(jnp.sqrt(jnp.sum(e0 * e0, axis=-1)), 1e-6)
    inve1 = 1.0 / jnp.maximum(jnp.sqrt(jnp.sum(e1 * e1, axis=-1)), 1e-6)
    sim0 = d0 * invp[...] * inve0[None]
    sim1 = d1 * invp[...] * inve1[None]
    w0 = jnp.where(validr, jax.nn.sigmoid(sim0 / 0.2) * ges, 0.0)
    w1 = jnp.where(validr, jax.nn.sigmoid(sim1 / 0.2) * ges, 0.0)

    @pl.when(j == 0)
    def _():
        an0[b] = w0 * sim0
        ad0[b] = w0
        an1[b] = w1 * sim1
        ad1[b] = w1

    @pl.when(j > 0)
    def _():
        an0[b] = an0[b] + w0 * sim0
        ad0[b] = ad0[b] + w0
        an1[b] = an1[b] + w1 * sim1
        ad1[b] = ad1[b] + w1

    @pl.when(j == _EB - 1)
    def _():
        nume_ref[0, 0] = jnp.sum(an0[b], axis=1, keepdims=True)
        nume_ref[0, 1] = jnp.sum(an1[b], axis=1, keepdims=True)
        dene_ref[0, 0] = jnp.sum(ad0[b], axis=1, keepdims=True)
        dene_ref[0, 1] = jnp.sum(ad1[b], axis=1, keepdims=True)


def _sc_scatter(m_hbm, dst3_hbm, zer_hbm, agg_hbm,
                dst_v, rows_v, z_v, acc_sh, lsem):
    c = lax.axis_index("c")
    s = lax.axis_index("s")
    pltpu.sync_copy(dst3_hbm.at[s], dst_v)
    pltpu.sync_copy(zer_hbm, z_v)

    def _pair(k, _):
        p = c * (_BP // 2) + k
        pltpu.sync_copy(z_v, acc_sh.at[pl.ds(s * _NPT, 128)])
        pltpu.sync_copy(z_v, acc_sh.at[pl.ds(s * _NPT + 128, 128)])
        plsc.subcore_barrier()

        def _group(g, _):
            hs = []
            for t in range(_SK):
                cc = g * _SK + t
                hs.append(pltpu.async_copy(
                    m_hbm.at[p, pl.ds(s * _EPW + cc * 128, 128), :],
                    rows_v.at[t], lsem))
            for t in range(_SK):
                cc = g * _SK + t
                hs[t].wait()
                pltpu.sync_copy(rows_v.at[t], acc_sh.at[dst_v.at[cc]], add=True)
            return 0

        lax.fori_loop(0, _SG, _group, 0)
        plsc.subcore_barrier()
        pltpu.sync_copy(acc_sh.at[pl.ds(s * _NPT, _NPT)],
                        agg_hbm.at[p, pl.ds(s * _NPT, _NPT), :])
        plsc.subcore_barrier()
        return 0

    lax.fori_loop(0, _BP // 2, _pair, 0)


def _stage3(h0_ref, agg_ref, wu_ref, bu_ref, g3_ref, b3_ref, pn_ref, gn_ref,
            numn_ref, denn_ref, invp, an0, ad0, an1, ad1):
    j = pl.program_id(0)
    b = pl.program_id(1)
    upd = jnp.dot(agg_ref[0], wu_ref[...], preferred_element_type=jnp.float32)
    hf = _ln2(h0_ref[...] + upd + bu_ref[...], g3_ref[...], b3_ref[...])
    pn = pn_ref[...]

    @pl.when(b == 0)
    def _():
        sp = jnp.sum(pn * pn, axis=-1)
        invp[...] = 1.0 / jnp.maximum(jnp.sqrt(sp), 1e-6)

    gns = jax.nn.sigmoid(gn_ref[...])
    h0 = hf[:, :_H]
    h1 = hf[:, _H:]
    d0 = jnp.sum(pn * h0[None], axis=-1)
    d1 = jnp.sum(pn * h1[None], axis=-1)
    invh0 = 1.0 / jnp.maximum(jnp.sqrt(jnp.sum(h0 * h0, axis=-1)), 1e-6)
    invh1 = 1.0 / jnp.maximum(jnp.sqrt(jnp.sum(h1 * h1, axis=-1)), 1e-6)
    sim0 = d0 * invp[...] * invh0[None]
    sim1 = d1 * invp[...] * invh1[None]
    w0 = jax.nn.sigmoid(sim0 / 0.2) * gns
    w1 = jax.nn.sigmoid(sim1 / 0.2) * gns

    @pl.when(j == 0)
    def _():
        an0[b] = w0 * sim0
        ad0[b] = w0
        an1[b] = w1 * sim1
        ad1[b] = w1

    @pl.when(j > 0)
    def _():
        an0[b] = an0[b] + w0 * sim0
        ad0[b] = ad0[b] + w0
        an1[b] = an1[b] + w1 * sim1
        ad1[b] = ad1[b] + w1

    @pl.when(j == _NB - 1)
    def _():
        numn_ref[0, 0] = jnp.sum(an0[b], axis=1, keepdims=True)
        numn_ref[0, 1] = jnp.sum(an1[b], axis=1, keepdims=True)
        denn_ref[0, 0] = jnp.sum(ad0[b], axis=1, keepdims=True)
        denn_ref[0, 1] = jnp.sum(ad1[b], axis=1, keepdims=True)


def _blkdiag(w):
    k, n = w.shape
    z = jnp.zeros((2 * k, 2 * n), w.dtype)
    return z.at[:k, :n].set(w).at[k:, n:].set(w)


def kernel(x, edge_index, edge_attr, W_node, b_node, ln1_g, ln1_b,
           W_edge, b_edge, ln2_g, ln2_b, W_msg, b_msg, W_upd, b_upd,
           ln3_g, ln3_b, proto_n, proto_e, gate_n, gate_e):
    f32 = jnp.float32
    pad_e = _EPAD - _E
    xp = jnp.concatenate([x[0::2], x[1::2]], axis=-1)
    ea = jnp.pad(edge_attr, ((0, 0), (0, pad_e), (0, 0)))
    ea_p = jnp.concatenate([ea[0::2], ea[1::2]], axis=-1)
    pe_p = jnp.pad(proto_e, ((0, 0), (0, pad_e), (0, 0)))
    ge_p = jnp.pad(gate_e, ((0, 0), (0, pad_e)))
    src_p = jnp.pad(edge_index[0], (0, pad_e))
    dst3 = jnp.pad(edge_index[1], (0, pad_e)).reshape(16, _EPW // 128, 128)
    zer = jnp.zeros((128, _H2), f32)
    wn2 = _blkdiag(W_node)
    we2 = _blkdiag(W_edge)
    wmt2 = _blkdiag(W_msg[:_H])
    wmb2 = _blkdiag(W_msg[_H:])
    wu2 = _blkdiag(W_upd)
    bn2 = jnp.tile(b_node, 2).reshape(1, _H2)
    be2 = jnp.tile(b_edge, 2).reshape(1, _H2)
    bm2 = jnp.tile(b_msg, 2).reshape(1, _H2)
    bu2 = jnp.tile(b_upd, 2).reshape(1, _H2)
    g1 = jnp.tile(ln1_g, 2).reshape(1, _H2)
    b1 = jnp.tile(ln1_b, 2).reshape(1, _H2)
    g2 = jnp.tile(ln2_g, 2).reshape(1, _H2)
    b2 = jnp.tile(ln2_b, 2).reshape(1, _H2)
    g3 = jnp.tile(ln3_g, 2).reshape(1, _H2)
    b3 = jnp.tile(ln3_b, 2).reshape(1, _H2)

    h0f = pl.pallas_call(
        _stage1,
        grid=(_BP, _NB),
        in_specs=[
            pl.BlockSpec((1, _NBLK, 2 * _ND), lambda b, j: (b, j, 0)),
            pl.BlockSpec((2 * _ND, _H2), lambda b, j: (0, 0)),
            pl.BlockSpec((1, _H2), lambda b, j: (0, 0)),
            pl.BlockSpec((1, _H2), lambda b, j: (0, 0)),
            pl.BlockSpec((1, _H2), lambda b, j: (0, 0)),
        ],
        out_specs=pl.BlockSpec((_NBLK, _H2), lambda b, j: (b * _NB + j, 0)),
        out_shape=jax.ShapeDtypeStruct((_BP * _N, _H2), f32),
    )(xp, wn2, bn2, g1, b1)

    mesh = plsc.VectorSubcoreMesh(core_axis_name="c", subcore_axis_name="s")
    h_src = pl.kernel(
        _sc_gather,
        mesh=mesh,
        out_type=jax.ShapeDtypeStruct((_BP, _EPAD, _H2), f32),
        scratch_types=[
            pltpu.VMEM((_EHW,), jnp.int32),
            pltpu.VMEM((_GK, 128, _H2), f32),
            pltpu.SemaphoreType.DMA,
            pltpu.SemaphoreType.DMA,
        ],
    )(h0f, src_p)

    m, num_e, den_e = pl.pallas_call(
        _stage2,
        grid=(_EB, _BP),
        in_specs=[
            pl.BlockSpec((1, _EBLK, 2 * _ED), lambda j, b: (b, j, 0)),
            pl.BlockSpec((1, _EBLK, _H2), lambda j, b: (b, j, 0)),
            pl.BlockSpec((_C, _EBLK, _H), lambda j, b: (0, j, 0)),
            pl.BlockSpec((_C, _EBLK), lambda j, b: (0, j)),
            pl.BlockSpec((2 * _ED, _H2), lambda j, b: (0, 0)),
            pl.BlockSpec((1, _H2), lambda j, b: (0, 0)),
            pl.BlockSpec((1, _H2), lambda j, b: (0, 0)),
            pl.BlockSpec((1, _H2), lambda j, b: (0, 0)),
            pl.BlockSpec((_H2, _H2), lambda j, b: (0, 0)),
            pl.BlockSpec((_H2, _H2), lambda j, b: (0, 0)),
            pl.BlockSpec((1, _H2), lambda j, b: (0, 0)),
        ],
        out_specs=[
            pl.BlockSpec((1, _EBLK, _H2), lambda j, b: (b, j, 0)),
            pl.BlockSpec((1, 2, _C, 1), lambda j, b: (b, 0, 0, 0)),
            pl.BlockSpec((1, 2, _C, 1), lambda j, b: (b, 0, 0, 0)),
        ],
        out_shape=[
            jax.ShapeDtypeStruct((_BP, _EPAD, _H2), f32),
            jax.ShapeDtypeStruct((_BP, 2, _C, 1), f32),
            jax.ShapeDtypeStruct((_BP, 2, _C, 1), f32),
        ],
        scratch_shapes=[
            pltpu.VMEM((_C, _EBLK), f32),
            pltpu.VMEM((_BP, _C, _EBLK), f32),
            pltpu.VMEM((_BP, _C, _EBLK), f32),
            pltpu.VMEM((_BP, _C, _EBLK), f32),
            pltpu.VMEM((_BP, _C, _EBLK), f32),
        ],
    )(ea_p, h_src, pe_p, ge_p, we2, be2, g2, b2, wmt2, wmb2, bm2)

    agg = pl.kernel(
        _sc_scatter,
        mesh=mesh,
        out_type=jax.ShapeDtypeStruct((_BP, _N, _H2), f32),
        scratch_types=[
            pltpu.VMEM((_EPW // 128, 128), jnp.int32),
            pltpu.VMEM((_SK, 128, _H2), f32),
            pltpu.VMEM((128, _H2), f32),
            pltpu.VMEM_SHARED((_N, _H2), f32),
            pltpu.SemaphoreType.DMA,
        ],
    )(m, dst3, zer)

    num_n, den_n = pl.pallas_call(
        _stage3,
        grid=(_NB, _BP),
        in_specs=[
            pl.BlockSpec((_NBLK, _H2), lambda j, b: (b * _NB + j, 0)),
            pl.BlockSpec((1, _NBLK, _H2), lambda j, b: (b, j, 0)),
            pl.BlockSpec((_H2, _H2), lambda j, b: (0, 0)),
            pl.BlockSpec((1, _H2), lambda j, b: (0, 0)),
            pl.BlockSpec((1, _H2), lambda j, b: (0, 0)),
            pl.BlockSpec((1, _H2), lambda j, b: (0, 0)),
            pl.BlockSpec((_C, _NBLK, _H), lambda j, b: (0, j, 0)),
            pl.BlockSpec((_C, _NBLK), lambda j, b: (0, j)),
        ],
        out_specs=[
            pl.BlockSpec((1, 2, _C, 1), lambda j, b: (b, 0, 0, 0)),
            pl.BlockSpec((1, 2, _C, 1), lambda j, b: (b, 0, 0, 0)),
        ],
        out_shape=[
            jax.ShapeDtypeStruct((_BP, 2, _C, 1), f32),
            jax.ShapeDtypeStruct((_BP, 2, _C, 1), f32),
        ],
        scratch_shapes=[
            pltpu.VMEM((_C, _NBLK), f32),
            pltpu.VMEM((_BP, _C, _NBLK), f32),
            pltpu.VMEM((_BP, _C, _NBLK), f32),
            pltpu.VMEM((_BP, _C, _NBLK), f32),
            pltpu.VMEM((_BP, _C, _NBLK), f32),
        ],
    )(h0f, agg, wu2, bu2, g3, b3, proto_n, gate_n)

    ns = num_n[..., 0].reshape(_B, _C) / jnp.maximum(
        den_n[..., 0].reshape(_B, _C), 1e-6)
    es = num_e[..., 0].reshape(_B, _C) / jnp.maximum(
        den_e[..., 0].reshape(_B, _C), 1e-6)
    return ns + 0.5 * es


# speedup vs baseline: 1.1849x; 1.1849x over previous
import jax
import jax.numpy as jnp
from jax import lax
from jax.experimental import pallas as pl
from jax.experimental.pallas import tpu as pltpu
from jax.experimental.pallas import tpu_sc as plsc

_B, _C, _N, _E = 16, 7, 4096, 32004
_ND, _ED, _H = 7, 5, 64
_BP = _B // 2
_H2 = 2 * _H
_EPAD = 32768
_EBLK = 1024
_EB = _EPAD // _EBLK
_NBLK = 1024
_NB = _N // _NBLK

_EHW = _EPAD // 4
_GK = 4
_GG = _EHW // (128 * _GK)
_EPW = _EPAD // 16
_SK = 4
_SG = _EPW // (128 * _SK)
_NPT = _N // 16


def _gelu(x):
    return 0.5 * x * (1.0 + jax.lax.erf(x * 0.7071067811865476))


def _ln(z, g, b):
    mu = jnp.mean(z, axis=-1, keepdims=True)
    var = jnp.mean((z - mu) ** 2, axis=-1, keepdims=True)
    return (z - mu) / jnp.sqrt(var + 1e-5) * g + b


def _ln2(z, g, b):
    return jnp.concatenate(
        [_ln(z[:, :_H], g[:, :_H], b[:, :_H]),
         _ln(z[:, _H:], g[:, _H:], b[:, _H:])], axis=-1)


def _stage1(x_ref, wn_ref, bn_ref, g1_ref, b1_ref, h_ref):
    z = jnp.dot(x_ref[0], wn_ref[...], preferred_element_type=jnp.float32)
    z = z + bn_ref[...]
    h_ref[...] = _gelu(_ln2(z, g1_ref[...], b1_ref[...]))


def _sc_gather(h0_hbm, src_hbm, out_hbm, idx_v, rows_v, gsem, osem):
    c = lax.axis_index("c")
    s = lax.axis_index("s")
    w = s * 2 + c
    p = w // 4
    quarter = w % 4
    base_e = quarter * _EHW
    pltpu.sync_copy(src_hbm.at[pl.ds(base_e, _EHW)], idx_v)
    pn = p * _N

    def _addbase(i, _):
        idx_v[pl.ds(i * 16, 16)] = idx_v[pl.ds(i * 16, 16)] + pn
        return 0

    lax.fori_loop(0, _EHW // 16, _addbase, 0)

    def _group(g, _):
        hs = []
        for k in range(_GK):
            j = g * _GK + k
            hs.append(pltpu.async_copy(
                h0_hbm.at[idx_v.at[pl.ds(j * 128, 128)]], rows_v.at[k], gsem))
        for k in range(_GK):
            hs[k].wait()
        os = []
        for k in range(_GK):
            j = g * _GK + k
            os.append(pltpu.async_copy(
                rows_v.at[k],
                out_hbm.at[p, pl.ds(base_e + j * 128, 128), :], osem))
        for k in range(_GK):
            os[k].wait()
        return 0

    lax.fori_loop(0, _GG, _group, 0)


def _stage2(ea_ref, hs_ref, pe_ref, ge_ref,
            we_ref, be_ref, g2_ref, b2_ref, wmt_ref, wmb_ref, bm_ref,
            m_ref, nume_ref, dene_ref, an0, ad0, an1, ad1):
    j = pl.program_id(1)
    z = jnp.dot(ea_ref[0], we_ref[...], preferred_element_type=jnp.float32)
    z = z + be_ref[...]
    e = _gelu(_ln2(z, g2_ref[...], b2_ref[...]))
    mpre = (jnp.dot(hs_ref[0], wmt_ref[...], preferred_element_type=jnp.float32)
            + jnp.dot(e, wmb_ref[...], preferred_element_type=jnp.float32)
            + bm_ref[...])
    m = _gelu(mpre)
    validc = jax.lax.broadcasted_iota(jnp.int32, (_EBLK, 1), 0) < (_E - j * _EBLK)
    m_ref[0] = jnp.where(validc, m, 0.0)

    pe = pe_ref[...]
    invp = 1.0 / jnp.maximum(
        jnp.sqrt(jnp.sum(pe * pe, axis=-1)), 1e-6)
    ges = jax.nn.sigmoid(ge_ref[...])
    validr = jax.lax.broadcasted_iota(
        jnp.int32, (1, _EBLK), 1) < (_E - j * _EBLK)

    e0 = e[:, :_H]
    e1 = e[:, _H:]
    d0 = jnp.sum(pe * e0[None], axis=-1)
    d1 = jnp.sum(pe * e1[None], axis=-1)
    inve0 = 1.0 / jnp.maximum(jnp.sqrt(jnp.sum(e0 * e0, axis=-1)), 1e-6)
    inve1 = 1.0 / jnp.maximum(jnp.sqrt(jnp.sum(e1 * e1, axis=-1)), 1e-6)
    sim0 = d0 * invp * inve0[None]
    sim1 = d1 * invp * inve1[None]
    w0 = jnp.where(validr, jax.nn.sigmoid(sim0 / 0.2) * ges, 0.0)
    w1 = jnp.where(validr, jax.nn.sigmoid(sim1 / 0.2) * ges, 0.0)

    @pl.when(j == 0)
    def _():
        an0[...] = w0 * sim0
        ad0[...] = w0
        an1[...] = w1 * sim1
        ad1[...] = w1

    @pl.when(j > 0)
    def _():
        an0[...] = an0[...] + w0 * sim0
        ad0[...] = ad0[...] + w0
        an1[...] = an1[...] + w1 * sim1
        ad1[...] = ad1[...] + w1

    @pl.when(j == _EB - 1)
    def _():
        nume_ref[0, 0] = jnp.sum(an0[...], axis=1, keepdims=True)
        nume_ref[0, 1] = jnp.sum(an1[...], axis=1, keepdims=True)
        dene_ref[0, 0] = jnp.sum(ad0[...], axis=1, keepdims=True)
        dene_ref[0, 1] = jnp.sum(ad1[...], axis=1, keepdims=True)


def _sc_scatter(m_hbm, dst3_hbm, zer_hbm, agg_hbm,
                dst_v, rows_v, z_v, acc_sh, lsem):
    c = lax.axis_index("c")
    s = lax.axis_index("s")
    pltpu.sync_copy(dst3_hbm.at[s], dst_v)
    pltpu.sync_copy(zer_hbm, z_v)

    def _pair(k, _):
        p = c * (_BP // 2) + k
        pltpu.sync_copy(z_v, acc_sh.at[pl.ds(s * _NPT, 128)])
        pltpu.sync_copy(z_v, acc_sh.at[pl.ds(s * _NPT + 128, 128)])
        plsc.subcore_barrier()

        def _group(g, _):
            hs = []
            for t in range(_SK):
                cc = g * _SK + t
                hs.append(pltpu.async_copy(
                    m_hbm.at[p, pl.ds(s * _EPW + cc * 128, 128), :],
                    rows_v.at[t], lsem))
            for t in range(_SK):
                cc = g * _SK + t
                hs[t].wait()
                pltpu.sync_copy(rows_v.at[t], acc_sh.at[dst_v.at[cc]], add=True)
            return 0

        lax.fori_loop(0, _SG, _group, 0)
        plsc.subcore_barrier()
        pltpu.sync_copy(acc_sh.at[pl.ds(s * _NPT, _NPT)],
                        agg_hbm.at[p, pl.ds(s * _NPT, _NPT), :])
        plsc.subcore_barrier()
        return 0

    lax.fori_loop(0, _BP // 2, _pair, 0)


def _stage3(h0_ref, agg_ref, wu_ref, bu_ref, g3_ref, b3_ref, pn_ref, gn_ref,
            numn_ref, denn_ref, an0, ad0, an1, ad1):
    j = pl.program_id(1)
    upd = jnp.dot(agg_ref[0], wu_ref[...], preferred_element_type=jnp.float32)
    hf = _ln2(h0_ref[...] + upd + bu_ref[...], g3_ref[...], b3_ref[...])
    pn = pn_ref[...]
    invp = 1.0 / jnp.maximum(
        jnp.sqrt(jnp.sum(pn * pn, axis=-1)), 1e-6)
    gns = jax.nn.sigmoid(gn_ref[...])
    h0 = hf[:, :_H]
    h1 = hf[:, _H:]
    d0 = jnp.sum(pn * h0[None], axis=-1)
    d1 = jnp.sum(pn * h1[None], axis=-1)
    invh0 = 1.0 / jnp.maximum(jnp.sqrt(jnp.sum(h0 * h0, axis=-1)), 1e-6)
    invh1 = 1.0 / jnp.maximum(jnp.sqrt(jnp.sum(h1 * h1, axis=-1)), 1e-6)
    sim0 = d0 * invp * invh0[None]
    sim1 = d1 * invp * invh1[None]
    w0 = jax.nn.sigmoid(sim0 / 0.2) * gns
    w1 = jax.nn.sigmoid(sim1 / 0.2) * gns

    @pl.when(j == 0)
    def _():
        an0[...] = w0 * sim0
        ad0[...] = w0
        an1[...] = w1 * sim1
        ad1[...] = w1

    @pl.when(j > 0)
    def _():
        an0[...] = an0[...] + w0 * sim0
        ad0[...] = ad0[...] + w0
        an1[...] = an1[...] + w1 * sim1
        ad1[...] = ad1[...] + w1

    @pl.when(j == _NB - 1)
    def _():
        numn_ref[0, 0] = jnp.sum(an0[...], axis=1, keepdims=True)
        numn_ref[0, 1] = jnp.sum(an1[...], axis=1, keepdims=True)
        denn_ref[0, 0] = jnp.sum(ad0[...], axis=1, keepdims=True)
        denn_ref[0, 1] = jnp.sum(ad1[...], axis=1, keepdims=True)


def _blkdiag(w):
    k, n = w.shape
    z = jnp.zeros((2 * k, 2 * n), w.dtype)
    return z.at[:k, :n].set(w).at[k:, n:].set(w)


def kernel(x, edge_index, edge_attr, W_node, b_node, ln1_g, ln1_b,
           W_edge, b_edge, ln2_g, ln2_b, W_msg, b_msg, W_upd, b_upd,
           ln3_g, ln3_b, proto_n, proto_e, gate_n, gate_e):
    f32 = jnp.float32
    pad_e = _EPAD - _E
    xp = jnp.concatenate([x[0::2], x[1::2]], axis=-1)
    ea = jnp.pad(edge_attr, ((0, 0), (0, pad_e), (0, 0)))
    ea_p = jnp.concatenate([ea[0::2], ea[1::2]], axis=-1)
    pe_p = jnp.pad(proto_e, ((0, 0), (0, pad_e), (0, 0)))
    ge_p = jnp.pad(gate_e, ((0, 0), (0, pad_e)))
    src_p = jnp.pad(edge_index[0], (0, pad_e))
    dst3 = jnp.pad(edge_index[1], (0, pad_e)).reshape(16, _EPW // 128, 128)
    zer = jnp.zeros((128, _H2), f32)
    wn2 = _blkdiag(W_node)
    we2 = _blkdiag(W_edge)
    wmt2 = _blkdiag(W_msg[:_H])
    wmb2 = _blkdiag(W_msg[_H:])
    wu2 = _blkdiag(W_upd)
    bn2 = jnp.tile(b_node, 2).reshape(1, _H2)
    be2 = jnp.tile(b_edge, 2).reshape(1, _H2)
    bm2 = jnp.tile(b_msg, 2).reshape(1, _H2)
    bu2 = jnp.tile(b_upd, 2).reshape(1, _H2)
    g1 = jnp.tile(ln1_g, 2).reshape(1, _H2)
    b1 = jnp.tile(ln1_b, 2).reshape(1, _H2)
    g2 = jnp.tile(ln2_g, 2).reshape(1, _H2)
    b2 = jnp.tile(ln2_b, 2).reshape(1, _H2)
    g3 = jnp.tile(ln3_g, 2).reshape(1, _H2)
    b3 = jnp.tile(ln3_b, 2).reshape(1, _H2)

    h0f = pl.pallas_call(
        _stage1,
        grid=(_BP, _NB),
        in_specs=[
            pl.BlockSpec((1, _NBLK, 2 * _ND), lambda b, j: (b, j, 0)),
            pl.BlockSpec((2 * _ND, _H2), lambda b, j: (0, 0)),
            pl.BlockSpec((1, _H2), lambda b, j: (0, 0)),
            pl.BlockSpec((1, _H2), lambda b, j: (0, 0)),
            pl.BlockSpec((1, _H2), lambda b, j: (0, 0)),
        ],
        out_specs=pl.BlockSpec((_NBLK, _H2), lambda b, j: (b * _NB + j, 0)),
        out_shape=jax.ShapeDtypeStruct((_BP * _N, _H2), f32),
    )(xp, wn2, bn2, g1, b1)

    mesh = plsc.VectorSubcoreMesh(core_axis_name="c", subcore_axis_name="s")
    h_src = pl.kernel(
        _sc_gather,
        mesh=mesh,
        out_type=jax.ShapeDtypeStruct((_BP, _EPAD, _H2), f32),
        scratch_types=[
            pltpu.VMEM((_EHW,), jnp.int32),
            pltpu.VMEM((_GK, 128, _H2), f32),
            pltpu.SemaphoreType.DMA,
            pltpu.SemaphoreType.DMA,
        ],
    )(h0f, src_p)

    m, num_e, den_e = pl.pallas_call(
        _stage2,
        grid=(_BP, _EB),
        in_specs=[
            pl.BlockSpec((1, _EBLK, 2 * _ED), lambda b, j: (b, j, 0)),
            pl.BlockSpec((1, _EBLK, _H2), lambda b, j: (b, j, 0)),
            pl.BlockSpec((_C, _EBLK, _H), lambda b, j: (0, j, 0)),
            pl.BlockSpec((_C, _EBLK), lambda b, j: (0, j)),
            pl.BlockSpec((2 * _ED, _H2), lambda b, j: (0, 0)),
            pl.BlockSpec((1, _H2), lambda b, j: (0, 0)),
            pl.BlockSpec((1, _H2), lambda b, j: (0, 0)),
            pl.BlockSpec((1, _H2), lambda b, j: (0, 0)),
            pl.BlockSpec((_H2, _H2), lambda b, j: (0, 0)),
            pl.BlockSpec((_H2, _H2), lambda b, j: (0, 0)),
            pl.BlockSpec((1, _H2), lambda b, j: (0, 0)),
        ],
        out_specs=[
            pl.BlockSpec((1, _EBLK, _H2), lambda b, j: (b, j, 0)),
            pl.BlockSpec((1, 2, _C, 1), lambda b, j: (b, 0, 0, 0)),
            pl.BlockSpec((1, 2, _C, 1), lambda b, j: (b, 0, 0, 0)),
        ],
        out_shape=[
            jax.ShapeDtypeStruct((_BP, _EPAD, _H2), f32),
            jax.ShapeDtypeStruct((_BP, 2, _C, 1), f32),
            jax.ShapeDtypeStruct((_BP, 2, _C, 1), f32),
        ],
        scratch_shapes=[
            pltpu.VMEM((_C, _EBLK), f32),
            pltpu.VMEM((_C, _EBLK), f32),
            pltpu.VMEM((_C, _EBLK), f32),
            pltpu.VMEM((_C, _EBLK), f32),
        ],
    )(ea_p, h_src, pe_p, ge_p, we2, be2, g2, b2, wmt2, wmb2, bm2)

    agg = pl.kernel(
        _sc_scatter,
        mesh=mesh,
        out_type=jax.ShapeDtypeStruct((_BP, _N, _H2), f32),
        scratch_types=[
            pltpu.VMEM((_EPW // 128, 128), jnp.int32),
            pltpu.VMEM((_SK, 128, _H2), f32),
            pltpu.VMEM((128, _H2), f32),
            pltpu.VMEM_SHARED((_N, _H2), f32),
            pltpu.SemaphoreType.DMA,
        ],
    )(m, dst3, zer)

    num_n, den_n = pl.pallas_call(
        _stage3,
        grid=(_BP, _NB),
        in_specs=[
            pl.BlockSpec((_NBLK, _H2), lambda b, j: (b * _NB + j, 0)),
            pl.BlockSpec((1, _NBLK, _H2), lambda b, j: (b, j, 0)),
            pl.BlockSpec((_H2, _H2), lambda b, j: (0, 0)),
            pl.BlockSpec((1, _H2), lambda b, j: (0, 0)),
            pl.BlockSpec((1, _H2), lambda b, j: (0, 0)),
            pl.BlockSpec((1, _H2), lambda b, j: (0, 0)),
            pl.BlockSpec((_C, _NBLK, _H), lambda b, j: (0, j, 0)),
            pl.BlockSpec((_C, _NBLK), lambda b, j: (0, j)),
        ],
        out_specs=[
            pl.BlockSpec((1, 2, _C, 1), lambda b, j: (b, 0, 0, 0)),
            pl.BlockSpec((1, 2, _C, 1), lambda b, j: (b, 0, 0, 0)),
        ],
        out_shape=[
            jax.ShapeDtypeStruct((_BP, 2, _C, 1), f32),
            jax.ShapeDtypeStruct((_BP, 2, _C, 1), f32),
        ],
        scratch_shapes=[
            pltpu.VMEM((_C, _NBLK), f32),
            pltpu.VMEM((_C, _NBLK), f32),
            pltpu.VMEM((_C, _NBLK), f32),
            pltpu.VMEM((_C, _NBLK), f32),
        ],
    )(h0f, agg, wu2, bu2, g3, b3, proto_n, gate_n)

    ns = num_n[..., 0].reshape(_B, _C) / jnp.maximum(
        den_n[..., 0].reshape(_B, _C), 1e-6)
    es = num_e[..., 0].reshape(_B, _C) / jnp.maximum(
        den_e[..., 0].reshape(_B, _C), 1e-6)
    return ns + 0.5 * es


# speedup vs baseline: 1.1922x; 1.0062x over previous
import jax
import jax.numpy as jnp
from jax import lax
from jax.experimental import pallas as pl
from jax.experimental.pallas import tpu as pltpu
from jax.experimental.pallas import tpu_sc as plsc

_B, _C, _N, _E = 16, 7, 4096, 32004
_ND, _ED, _H = 7, 5, 64
_BP = _B // 2
_H2 = 2 * _H
_EPAD = 32768
_EBLK = 1024
_EB = _EPAD // _EBLK
_NBLK = 1024
_NB = _N // _NBLK

_EHW = _EPAD // 4
_GK = 4
_GG = _EHW // (128 * _GK)
_EPW = _EPAD // 16
_SK = 4
_SG = _EPW // (128 * _SK)
_NPT = _N // 16


def _gelu(x):
    return 0.5 * x * (1.0 + jax.lax.erf(x * 0.7071067811865476))


def _ln(z, g, b):
    mu = jnp.mean(z, axis=-1, keepdims=True)
    var = jnp.mean((z - mu) ** 2, axis=-1, keepdims=True)
    return (z - mu) / jnp.sqrt(var + 1e-5) * g + b


def _ln2(z, g, b):
    return jnp.concatenate(
        [_ln(z[:, :_H], g[:, :_H], b[:, :_H]),
         _ln(z[:, _H:], g[:, _H:], b[:, _H:])], axis=-1)


def _stage1(x_ref, wn_ref, bn_ref, g1_ref, b1_ref, h_ref):
    z = jnp.dot(x_ref[0], wn_ref[...], preferred_element_type=jnp.float32)
    z = z + bn_ref[...]
    h_ref[...] = _gelu(_ln2(z, g1_ref[...], b1_ref[...]))


def _sc_gather(h0_hbm, src_hbm, out_hbm, idx_v, rows_v, gsem, osem):
    c = lax.axis_index("c")
    s = lax.axis_index("s")
    w = s * 2 + c
    p = w // 4
    quarter = w % 4
    base_e = quarter * _EHW
    pltpu.sync_copy(src_hbm.at[pl.ds(base_e, _EHW)], idx_v)
    pn = p * _N

    def _addbase(i, _):
        idx_v[pl.ds(i * 16, 16)] = idx_v[pl.ds(i * 16, 16)] + pn
        return 0

    lax.fori_loop(0, _EHW // 16, _addbase, 0)

    def _group(g, _):
        hs = []
        for k in range(_GK):
            j = g * _GK + k
            hs.append(pltpu.async_copy(
                h0_hbm.at[idx_v.at[pl.ds(j * 128, 128)]], rows_v.at[k], gsem))
        for k in range(_GK):
            hs[k].wait()
        os = []
        for k in range(_GK):
            j = g * _GK + k
            os.append(pltpu.async_copy(
                rows_v.at[k],
                out_hbm.at[p, pl.ds(base_e + j * 128, 128), :], osem))
        for k in range(_GK):
            os[k].wait()
        return 0

    lax.fori_loop(0, _GG, _group, 0)


def _stage2(ea_ref, hs_ref, pe_ref, ge_ref,
            we_ref, be_ref, g2_ref, b2_ref, wmt_ref, wmb_ref, bm_ref,
            m_ref, nume_ref, dene_ref, an0, ad0, an1, ad1):
    j = pl.program_id(1)
    z = jnp.dot(ea_ref[0], we_ref[...], preferred_element_type=jnp.float32)
    z = z + be_ref[...]
    e = _gelu(_ln2(z, g2_ref[...], b2_ref[...]))
    mpre = (jnp.dot(hs_ref[0], wmt_ref[...], preferred_element_type=jnp.float32)
            + jnp.dot(e, wmb_ref[...], preferred_element_type=jnp.float32)
            + bm_ref[...])
    m = _gelu(mpre)
    validc = jax.lax.broadcasted_iota(jnp.int32, (_EBLK, 1), 0) < (_E - j * _EBLK)
    m_ref[0] = jnp.where(validc, m, 0.0)

    pe = pe_ref[...]
    invp = 1.0 / jnp.maximum(
        jnp.sqrt(jnp.sum(pe * pe, axis=-1)), 1e-6)
    ges = jax.nn.sigmoid(ge_ref[...])
    validr = jax.lax.broadcasted_iota(
        jnp.int32, (1, _EBLK), 1) < (_E - j * _EBLK)

    e0 = e[:, :_H]
    e1 = e[:, _H:]
    d0 = jnp.sum(pe * e0[None], axis=-1)
    d1 = jnp.sum(pe * e1[None], axis=-1)
    inve0 = 1.0 / jnp.maximum(jnp.sqrt(jnp.sum(e0 * e0, axis=-1)), 1e-6)
    inve1 = 1.0 / jnp.maximum(jnp.sqrt(jnp.sum(e1 * e1, axis=-1)), 1e-6)
    sim0 = d0 * invp * inve0[None]
    sim1 = d1 * invp * inve1[None]
    w0 = jnp.where(validr, jax.nn.sigmoid(sim0 / 0.2) * ges, 0.0)
    w1 = jnp.where(validr, jax.nn.sigmoid(sim1 / 0.2) * ges, 0.0)
    pn0 = jnp.sum((w0 * sim0).reshape(_C, _EBLK // 128, 128), axis=1)
    pd0 = jnp.sum(w0.reshape(_C, _EBLK // 128, 128), axis=1)
    pn1 = jnp.sum((w1 * sim1).reshape(_C, _EBLK // 128, 128), axis=1)
    pd1 = jnp.sum(w1.reshape(_C, _EBLK // 128, 128), axis=1)

    @pl.when(j == 0)
    def _():
        an0[...] = pn0
        ad0[...] = pd0
        an1[...] = pn1
        ad1[...] = pd1

    @pl.when(j > 0)
    def _():
        an0[...] = an0[...] + pn0
        ad0[...] = ad0[...] + pd0
        an1[...] = an1[...] + pn1
        ad1[...] = ad1[...] + pd1

    @pl.when(j == _EB - 1)
    def _():
        nume_ref[0, 0] = jnp.sum(an0[...], axis=1, keepdims=True)
        nume_ref[0, 1] = jnp.sum(an1[...], axis=1, keepdims=True)
        dene_ref[0, 0] = jnp.sum(ad0[...], axis=1, keepdims=True)
        dene_ref[0, 1] = jnp.sum(ad1[...], axis=1, keepdims=True)


def _sc_scatter(m_hbm, dst3_hbm, zer_hbm, agg_hbm,
                dst_v, rows_v, z_v, acc_sh, lsem):
    c = lax.axis_index("c")
    s = lax.axis_index("s")
    pltpu.sync_copy(dst3_hbm.at[s], dst_v)
    pltpu.sync_copy(zer_hbm, z_v)

    def _pair(k, _):
        p = c * (_BP // 2) + k
        pltpu.sync_copy(z_v, acc_sh.at[pl.ds(s * _NPT, 128)])
        pltpu.sync_copy(z_v, acc_sh.at[pl.ds(s * _NPT + 128, 128)])
        plsc.subcore_barrier()

        def _group(g, _):
            hs = []
            for t in range(_SK):
                cc = g * _SK + t
                hs.append(pltpu.async_copy(
                    m_hbm.at[p, pl.ds(s * _EPW + cc * 128, 128), :],
                    rows_v.at[t], lsem))
            for t in range(_SK):
                cc = g * _SK + t
                hs[t].wait()
                pltpu.sync_copy(rows_v.at[t], acc_sh.at[dst_v.at[cc]], add=True)
            return 0

        lax.fori_loop(0, _SG, _group, 0)
        plsc.subcore_barrier()
        pltpu.sync_copy(acc_sh.at[pl.ds(s * _NPT, _NPT)],
                        agg_hbm.at[p, pl.ds(s * _NPT, _NPT), :])
        plsc.subcore_barrier()
        return 0

    lax.fori_loop(0, _BP // 2, _pair, 0)


def _stage3(h0_ref, agg_ref, wu_ref, bu_ref, g3_ref, b3_ref, pn_ref, gn_ref,
            numn_ref, denn_ref, an0, ad0, an1, ad1):
    j = pl.program_id(1)
    upd = jnp.dot(agg_ref[0], wu_ref[...], preferred_element_type=jnp.float32)
    hf = _ln2(h0_ref[...] + upd + bu_ref[...], g3_ref[...], b3_ref[...])
    pn = pn_ref[...]
    invp = 1.0 / jnp.maximum(
        jnp.sqrt(jnp.sum(pn * pn, axis=-1)), 1e-6)
    gns = jax.nn.sigmoid(gn_ref[...])
    h0 = hf[:, :_H]
    h1 = hf[:, _H:]
    d0 = jnp.sum(pn * h0[None], axis=-1)
    d1 = jnp.sum(pn * h1[None], axis=-1)
    invh0 = 1.0 / jnp.maximum(jnp.sqrt(jnp.sum(h0 * h0, axis=-1)), 1e-6)
    invh1 = 1.0 / jnp.maximum(jnp.sqrt(jnp.sum(h1 * h1, axis=-1)), 1e-6)
    sim0 = d0 * invp * invh0[None]
    sim1 = d1 * invp * invh1[None]
    w0 = jax.nn.sigmoid(sim0 / 0.2) * gns
    w1 = jax.nn.sigmoid(sim1 / 0.2) * gns
    pn0 = jnp.sum((w0 * sim0).reshape(_C, _NBLK // 128, 128), axis=1)
    pd0 = jnp.sum(w0.reshape(_C, _NBLK // 128, 128), axis=1)
    pn1 = jnp.sum((w1 * sim1).reshape(_C, _NBLK // 128, 128), axis=1)
    pd1 = jnp.sum(w1.reshape(_C, _NBLK // 128, 128), axis=1)

    @pl.when(j == 0)
    def _():
        an0[...] = pn0
        ad0[...] = pd0
        an1[...] = pn1
        ad1[...] = pd1

    @pl.when(j > 0)
    def _():
        an0[...] = an0[...] + pn0
        ad0[...] = ad0[...] + pd0
        an1[...] = an1[...] + pn1
        ad1[...] = ad1[...] + pd1

    @pl.when(j == _NB - 1)
    def _():
        numn_ref[0, 0] = jnp.sum(an0[...], axis=1, keepdims=True)
        numn_ref[0, 1] = jnp.sum(an1[...], axis=1, keepdims=True)
        denn_ref[0, 0] = jnp.sum(ad0[...], axis=1, keepdims=True)
        denn_ref[0, 1] = jnp.sum(ad1[...], axis=1, keepdims=True)


def _blkdiag(w):
    k, n = w.shape
    z = jnp.zeros((2 * k, 2 * n), w.dtype)
    return z.at[:k, :n].set(w).at[k:, n:].set(w)


def kernel(x, edge_index, edge_attr, W_node, b_node, ln1_g, ln1_b,
           W_edge, b_edge, ln2_g, ln2_b, W_msg, b_msg, W_upd, b_upd,
           ln3_g, ln3_b, proto_n, proto_e, gate_n, gate_e):
    f32 = jnp.float32
    pad_e = _EPAD - _E
    xp = jnp.concatenate([x[0::2], x[1::2]], axis=-1)
    ea = jnp.pad(edge_attr, ((0, 0), (0, pad_e), (0, 0)))
    ea_p = jnp.concatenate([ea[0::2], ea[1::2]], axis=-1)
    pe_p = jnp.pad(proto_e, ((0, 0), (0, pad_e), (0, 0)))
    ge_p = jnp.pad(gate_e, ((0, 0), (0, pad_e)))
    src_p = jnp.pad(edge_index[0], (0, pad_e))
    dst3 = jnp.pad(edge_index[1], (0, pad_e)).reshape(16, _EPW // 128, 128)
    zer = jnp.zeros((128, _H2), f32)
    wn2 = _blkdiag(W_node)
    we2 = _blkdiag(W_edge)
    wmt2 = _blkdiag(W_msg[:_H])
    wmb2 = _blkdiag(W_msg[_H:])
    wu2 = _blkdiag(W_upd)
    bn2 = jnp.tile(b_node, 2).reshape(1, _H2)
    be2 = jnp.tile(b_edge, 2).reshape(1, _H2)
    bm2 = jnp.tile(b_msg, 2).reshape(1, _H2)
    bu2 = jnp.tile(b_upd, 2).reshape(1, _H2)
    g1 = jnp.tile(ln1_g, 2).reshape(1, _H2)
    b1 = jnp.tile(ln1_b, 2).reshape(1, _H2)
    g2 = jnp.tile(ln2_g, 2).reshape(1, _H2)
    b2 = jnp.tile(ln2_b, 2).reshape(1, _H2)
    g3 = jnp.tile(ln3_g, 2).reshape(1, _H2)
    b3 = jnp.tile(ln3_b, 2).reshape(1, _H2)

    h0f = pl.pallas_call(
        _stage1,
        grid=(_BP, _NB),
        in_specs=[
            pl.BlockSpec((1, _NBLK, 2 * _ND), lambda b, j: (b, j, 0)),
            pl.BlockSpec((2 * _ND, _H2), lambda b, j: (0, 0)),
            pl.BlockSpec((1, _H2), lambda b, j: (0, 0)),
            pl.BlockSpec((1, _H2), lambda b, j: (0, 0)),
            pl.BlockSpec((1, _H2), lambda b, j: (0, 0)),
        ],
        out_specs=pl.BlockSpec((_NBLK, _H2), lambda b, j: (b * _NB + j, 0)),
        out_shape=jax.ShapeDtypeStruct((_BP * _N, _H2), f32),
    )(xp, wn2, bn2, g1, b1)

    mesh = plsc.VectorSubcoreMesh(core_axis_name="c", subcore_axis_name="s")
    h_src = pl.kernel(
        _sc_gather,
        mesh=mesh,
        out_type=jax.ShapeDtypeStruct((_BP, _EPAD, _H2), f32),
        scratch_types=[
            pltpu.VMEM((_EHW,), jnp.int32),
            pltpu.VMEM((_GK, 128, _H2), f32),
            pltpu.SemaphoreType.DMA,
            pltpu.SemaphoreType.DMA,
        ],
    )(h0f, src_p)

    m, num_e, den_e = pl.pallas_call(
        _stage2,
        grid=(_BP, _EB),
        in_specs=[
            pl.BlockSpec((1, _EBLK, 2 * _ED), lambda b, j: (b, j, 0)),
            pl.BlockSpec((1, _EBLK, _H2), lambda b, j: (b, j, 0)),
            pl.BlockSpec((_C, _EBLK, _H), lambda b, j: (0, j, 0)),
            pl.BlockSpec((_C, _EBLK), lambda b, j: (0, j)),
            pl.BlockSpec((2 * _ED, _H2), lambda b, j: (0, 0)),
            pl.BlockSpec((1, _H2), lambda b, j: (0, 0)),
            pl.BlockSpec((1, _H2), lambda b, j: (0, 0)),
            pl.BlockSpec((1, _H2), lambda b, j: (0, 0)),
            pl.BlockSpec((_H2, _H2), lambda b, j: (0, 0)),
            pl.BlockSpec((_H2, _H2), lambda b, j: (0, 0)),
            pl.BlockSpec((1, _H2), lambda b, j: (0, 0)),
        ],
        out_specs=[
            pl.BlockSpec((1, _EBLK, _H2), lambda b, j: (b, j, 0)),
            pl.BlockSpec((1, 2, _C, 1), lambda b, j: (b, 0, 0, 0)),
            pl.BlockSpec((1, 2, _C, 1), lambda b, j: (b, 0, 0, 0)),
        ],
        out_shape=[
            jax.ShapeDtypeStruct((_BP, _EPAD, _H2), f32),
            jax.ShapeDtypeStruct((_BP, 2, _C, 1), f32),
            jax.ShapeDtypeStruct((_BP, 2, _C, 1), f32),
        ],
        scratch_shapes=[
            pltpu.VMEM((_C, 128), f32),
            pltpu.VMEM((_C, 128), f32),
            pltpu.VMEM((_C, 128), f32),
            pltpu.VMEM((_C, 128), f32),
        ],
    )(ea_p, h_src, pe_p, ge_p, we2, be2, g2, b2, wmt2, wmb2, bm2)

    agg = pl.kernel(
        _sc_scatter,
        mesh=mesh,
        out_type=jax.ShapeDtypeStruct((_BP, _N, _H2), f32),
        scratch_types=[
            pltpu.VMEM((_EPW // 128, 128), jnp.int32),
            pltpu.VMEM((_SK, 128, _H2), f32),
            pltpu.VMEM((128, _H2), f32),
            pltpu.VMEM_SHARED((_N, _H2), f32),
            pltpu.SemaphoreType.DMA,
        ],
    )(m, dst3, zer)

    num_n, den_n = pl.pallas_call(
        _stage3,
        grid=(_BP, _NB),
        in_specs=[
            pl.BlockSpec((_NBLK, _H2), lambda b, j: (b * _NB + j, 0)),
            pl.BlockSpec((1, _NBLK, _H2), lambda b, j: (b, j, 0)),
            pl.BlockSpec((_H2, _H2), lambda b, j: (0, 0)),
            pl.BlockSpec((1, _H2), lambda b, j: (0, 0)),
            pl.BlockSpec((1, _H2), lambda b, j: (0, 0)),
            pl.BlockSpec((1, _H2), lambda b, j: (0, 0)),
            pl.BlockSpec((_C, _NBLK, _H), lambda b, j: (0, j, 0)),
            pl.BlockSpec((_C, _NBLK), lambda b, j: (0, j)),
        ],
        out_specs=[
            pl.BlockSpec((1, 2, _C, 1), lambda b, j: (b, 0, 0, 0)),
            pl.BlockSpec((1, 2, _C, 1), lambda b, j: (b, 0, 0, 0)),
        ],
        out_shape=[
            jax.ShapeDtypeStruct((_BP, 2, _C, 1), f32),
            jax.ShapeDtypeStruct((_BP, 2, _C, 1), f32),
        ],
        scratch_shapes=[
            pltpu.VMEM((_C, 128), f32),
            pltpu.VMEM((_C, 128), f32),
            pltpu.VMEM((_C, 128), f32),
            pltpu.VMEM((_C, 128), f32),
        ],
    )(h0f, agg, wu2, bu2, g3, b3, proto_n, gate_n)

    ns = num_n[..., 0].reshape(_B, _C) / jnp.maximum(
        den_n[..., 0].reshape(_B, _C), 1e-6)
    es = num_e[..., 0].reshape(_B, _C) / jnp.maximum(
        den_e[..., 0].reshape(_B, _C), 1e-6)
    return ns + 0.5 * es
